# baseline (speedup 1.0000x reference)
import functools

import jax
import jax.numpy as jnp
from jax import lax
from jax.experimental import pallas as pl
from jax.experimental.pallas import tpu as pltpu
from jax.experimental.pallas import tpu_sc as plsc

N_NODES = 100000
D_FEAT = 1433
B = 256
S = 128
H = 64

_NT = 11
_TAIL = D_FEAT - 128 * _NT
_RTOT = B * S + B
_PW = _RTOT // 32
_CH = 64
_NCH = 16
_PRE_R = 1024


def _tail_body(emb, wtail, v_ref):
    x = emb[...]
    lane = lax.broadcasted_iota(jnp.int32, x.shape, 1)
    x = jnp.where(lane < _TAIL, x, 0.0)
    v = jnp.dot(x, wtail[...], preferred_element_type=jnp.float32)
    v_ref[...] = jnp.concatenate([v, v], axis=1)


def _tail_prepass(emb, wtail):
    n_steps = (N_NODES + _PRE_R - 1) // _PRE_R
    return pl.pallas_call(
        _tail_body,
        grid=(n_steps,),
        in_specs=[
            pl.BlockSpec((_PRE_R, 128), lambda i: (i, _NT)),
            pl.BlockSpec((128, H), lambda i: (0, 0)),
        ],
        out_specs=pl.BlockSpec((_PRE_R, 2 * H), lambda i: (i, 0)),
        out_shape=jax.ShapeDtypeStruct((N_NODES, 2 * H), jnp.float32),
    )(emb, wtail)


def _sc_gather_slabs(all_idx, emb):
    info = plsc.get_sparse_core_info()
    nc, ns = info.num_cores, info.num_subcores
    mesh = plsc.VectorSubcoreMesh(core_axis_name="c", subcore_axis_name="s")

    @functools.partial(
        pl.kernel,
        mesh=mesh,
        out_type=jax.ShapeDtypeStruct((_NT, _RTOT, 128), jnp.float32),
        scratch_types=[
            pltpu.VMEM((_PW,), jnp.int32),
            pltpu.VMEM((_NT, _CH, 128), jnp.float32),
            pltpu.SemaphoreType.DMA,
        ],
    )
    def k(aidx_hbm, emb_hbm, g_hbm, idx_v, buf, sem):
        wid = lax.axis_index("s") * nc + lax.axis_index("c")
        wbase = wid * _PW
        pltpu.sync_copy(aidx_hbm.at[pl.ds(wbase, _PW)], idx_v)

        def chunk(row0, rows):
            idx = idx_v.at[pl.ds(row0, rows)]
            cps = [
                pltpu.async_copy(
                    emb_hbm.at[idx, pl.ds(t * 128, 128)],
                    buf.at[t, pl.ds(0, rows)], sem)
                for t in range(_NT)
            ]
            for cp in cps:
                cp.wait()
            pltpu.sync_copy(buf.at[:, pl.ds(0, rows)],
                            g_hbm.at[:, pl.ds(wbase + row0, rows), :])

        def body(ci, carry):
            chunk(ci * _CH, _CH)
            return carry

        lax.fori_loop(0, _NCH, body, 0)
        chunk(_NCH * _CH, _PW - _NCH * _CH)

    return k(all_idx, emb)


def _sc_gather_tail(all_idx, v2):
    info = plsc.get_sparse_core_info()
    nc, ns = info.num_cores, info.num_subcores
    mesh = plsc.VectorSubcoreMesh(core_axis_name="c", subcore_axis_name="s")
    chv = 2 * _CH

    @functools.partial(
        pl.kernel,
        mesh=mesh,
        out_type=jax.ShapeDtypeStruct((_RTOT, 128), jnp.float32),
        scratch_types=[
            pltpu.VMEM((_PW,), jnp.int32),
            pltpu.VMEM((chv, 128), jnp.float32),
            pltpu.SemaphoreType.DMA,
        ],
    )
    def k(aidx_hbm, v_hbm, gt_hbm, idx_v, buf, sem):
        wid = lax.axis_index("s") * nc + lax.axis_index("c")
        wbase = wid * _PW
        pltpu.sync_copy(aidx_hbm.at[pl.ds(wbase, _PW)], idx_v)

        def chunk(row0, rows):
            pltpu.async_copy(
                v_hbm.at[idx_v.at[pl.ds(row0, rows)]],
                buf.at[pl.ds(0, rows)], sem).wait()
            pltpu.sync_copy(buf.at[pl.ds(0, rows)],
                            gt_hbm.at[pl.ds(wbase + row0, rows)])

        def body(ci, carry):
            chunk(ci * chv, chv)
            return carry

        lax.fori_loop(0, _PW // chv, body, 0)
        chunk(_PW - _PW % chv, _PW % chv)

    return k(all_idx, v2)


def _consume_body(g, gt, w1tp, b1, w2t, b2, g1t, a1, packed_ref, mal_ref):
    mm = lambda x, y: jnp.dot(x, y, preferred_element_type=jnp.float32)
    acc = gt[...][:, :H]
    for t in range(_NT):
        acc = acc + mm(g[t], w1tp[t])
    h = jnp.maximum(acc + b1[...], 0.0)
    y = mm(h, w2t[...]) + b2[...]
    z = mm(y, g1t[...])
    z = jnp.where(z >= 0, z, a1[0, 0] * z)

    z3 = z[:8 * S].reshape(8, S, H)
    zroot = z3[:, 0, :]
    pooled = (jnp.sum(z3, axis=1) - zroot) * (1.0 / (S - 1))
    packed_ref[...] = jnp.concatenate([pooled, zroot], axis=1)
    ym = y[8 * S:_PW]
    mal_ref[...] = jnp.concatenate([ym, ym], axis=1)


def _consume(g, gt, w1tp, b1, w2t, b2, g1t, a1):
    full = lambda i: (0, 0)
    out_sds = jax.ShapeDtypeStruct((B, 2 * H), jnp.float32)
    return pl.pallas_call(
        _consume_body,
        grid=(32,),
        in_specs=[
            pl.BlockSpec((_NT, _PW, 128), lambda i: (0, i, 0)),
            pl.BlockSpec((_PW, 128), lambda i: (i, 0)),
            pl.BlockSpec((_NT, 128, H), lambda i: (0, 0, 0)),
            pl.BlockSpec((1, H), full),
            pl.BlockSpec((H, H), full),
            pl.BlockSpec((1, H), full),
            pl.BlockSpec((H, H), full),
            pl.BlockSpec((1, 1), full),
        ],
        out_specs=[
            pl.BlockSpec((8, 2 * H), lambda i: (i, 0)),
            pl.BlockSpec((8, 2 * H), lambda i: (i, 0)),
        ],
        out_shape=[out_sds, out_sds],
    )(g, gt, w1tp, b1, w2t, b2, g1t, a1)


def _score_body(packed, mal, g1t, a1, g2t, a2, vn, vn1, sn, sn1,
                nwt, nb, n1wt, n1b, w1, c1, w2, c2, w3, c3,
                ps_ref, nps_ref, rs_ref, nrs_ref, ms_ref, pool_ref):
    mm = lambda x, y: jnp.dot(x, y, preferred_element_type=jnp.float32)
    pz = packed[...]
    p = pz[:, :H]
    zroot = pz[:, H:]
    m = mal[...][:, H:]

    root = mm(zroot, g2t[...])
    root = jnp.where(root >= 0, root, a2[0, 0] * root)

    vn1h = mm(vn1[...], g1t[...])
    vn1h = jnp.where(vn1h >= 0, vn1h, a1[0, 0] * vn1h)
    u1 = mm(vn1h, w1[...])
    vnh = mm(vn[...], g2t[...])
    vnh = jnp.where(vnh >= 0, vnh, a2[0, 0] * vnh)
    u2 = mm(vnh, w2[...])
    u3 = mm(vnh, w3[...])
    noise = mm(sn1[...], nwt[...]) + nb[...]
    rnoise = mm(sn[...], n1wt[...]) + n1b[...]

    ps = jnp.sum(p * u1, axis=1, keepdims=True) + c1[0, 0]
    ps_ref[...] = ps
    nps_ref[...] = ps + jnp.sum(noise * u1)
    rs = jnp.sum(root * u2, axis=1, keepdims=True) + c2[0, 0]
    rs_ref[...] = rs
    nrs_ref[...] = rs + jnp.sum(rnoise * u2)
    ms_ref[...] = jnp.sum((root + m) * 0.5 * u3, axis=1, keepdims=True) + c3[0, 0]
    pool_ref[...] = p


def _scores(packed, mal, g1t, a1, g2t, a2, vn, vn1, sn, sn1,
            nwt, nb, n1wt, n1b, w1, c1, w2, c2, w3, c3):
    s1 = jax.ShapeDtypeStruct((B, 1), jnp.float32)
    s64 = jax.ShapeDtypeStruct((B, H), jnp.float32)
    return pl.pallas_call(
        _score_body,
        out_shape=[s1, s1, s1, s1, s1, s64],
    )(packed, mal, g1t, a1, g2t, a2, vn, vn1, sn, sn1,
      nwt, nb, n1wt, n1b, w1, c1, w2, c2, w3, c3)


def kernel(subgraph_nodes, edge_index, malicious_nodes, embeddings, fe_W1, fe_b1, fe_W2, fe_b2, g1_W, a1, g2_W, a2, virtual_node, virtual_node1, single_noise, single_noise1, noise_W, noise_b, noise1_W, noise1_b, bil1_W, bil1_b, bil2_W, bil2_b, bil3_W, bil3_b):
    nodes = subgraph_nodes.astype(jnp.int32)
    mal_idx = malicious_nodes.astype(jnp.int32)
    all_idx = jnp.concatenate(
        [nodes.reshape(32, 8 * S), mal_idx.reshape(32, 8)], axis=1).reshape(-1)

    a1r = a1.reshape(1, 1)
    a2r = a2.reshape(1, 1)
    wt = fe_W1.T
    w1tp = wt[:128 * _NT].reshape(_NT, 128, H)
    wtail = jnp.pad(wt[128 * _NT:], ((0, 128 - _TAIL), (0, 0)))

    g = _sc_gather_slabs(all_idx, embeddings)
    v2 = _tail_prepass(embeddings, wtail)
    gt = _sc_gather_tail(all_idx, v2)
    packed, mal = _consume(g, gt, w1tp, fe_b1.reshape(1, H), fe_W2.T,
                           fe_b2.reshape(1, H), g1_W.T, a1r)

    ps, nps, rs, nrs, ms, pooled = _scores(
        packed, mal, g1_W.T, a1r, g2_W.T, a2r,
        virtual_node, virtual_node1, single_noise, single_noise1,
        noise_W.T, noise_b.reshape(1, H), noise1_W.T, noise1_b.reshape(1, H),
        bil1_W[0], bil1_b.reshape(1, 1), bil2_W[0], bil2_b.reshape(1, 1),
        bil3_W[0], bil3_b.reshape(1, 1))
    return (ps, nps, rs, nrs, ms, pooled)


# speedup vs baseline: 1.0009x; 1.0009x over previous
import functools

import jax
import jax.numpy as jnp
from jax import lax
from jax.experimental import pallas as pl
from jax.experimental.pallas import tpu as pltpu
from jax.experimental.pallas import tpu_sc as plsc

N_NODES = 100000
D_FEAT = 1433
B = 256
S = 128
H = 64

_NT = 11
_TAIL = D_FEAT - 128 * _NT
_RTOT = B * S + B
_PW = _RTOT // 32
_CH = 64
_NCH = 16
_PRE_R = 1024


def _tail_body(emb, wtail, v_ref):
    x = emb[...]
    lane = lax.broadcasted_iota(jnp.int32, x.shape, 1)
    x = jnp.where(lane < _TAIL, x, 0.0)
    v = jnp.dot(x, wtail[...], preferred_element_type=jnp.float32)
    v_ref[...] = jnp.concatenate([v, v], axis=1)


def _tail_prepass(emb, wtail):
    n_steps = (N_NODES + _PRE_R - 1) // _PRE_R
    return pl.pallas_call(
        _tail_body,
        grid=(n_steps,),
        in_specs=[
            pl.BlockSpec((_PRE_R, 128), lambda i: (i, _NT)),
            pl.BlockSpec((128, H), lambda i: (0, 0)),
        ],
        out_specs=pl.BlockSpec((_PRE_R, 2 * H), lambda i: (i, 0)),
        out_shape=jax.ShapeDtypeStruct((N_NODES, 2 * H), jnp.float32),
    )(emb, wtail)


def _sc_gather_slabs(all_idx, emb):
    info = plsc.get_sparse_core_info()
    nc, ns = info.num_cores, info.num_subcores
    mesh = plsc.VectorSubcoreMesh(core_axis_name="c", subcore_axis_name="s")

    @functools.partial(
        pl.kernel,
        mesh=mesh,
        compiler_params=pltpu.CompilerParams(use_tc_tiling_on_sc=True),
        out_type=jax.ShapeDtypeStruct((_NT, _RTOT, 128), jnp.float32),
        scratch_types=[
            pltpu.VMEM((_PW,), jnp.int32),
            pltpu.VMEM((_NT, _CH, 128), jnp.float32),
            pltpu.SemaphoreType.DMA,
        ],
    )
    def k(aidx_hbm, emb_hbm, g_hbm, idx_v, buf, sem):
        wid = lax.axis_index("s") * nc + lax.axis_index("c")
        wbase = wid * _PW
        pltpu.sync_copy(aidx_hbm.at[pl.ds(wbase, _PW)], idx_v)

        def chunk(row0, rows):
            idx = idx_v.at[pl.ds(row0, rows)]
            cps = [
                pltpu.async_copy(
                    emb_hbm.at[idx, pl.ds(t * 128, 128)],
                    buf.at[t, pl.ds(0, rows)], sem)
                for t in range(_NT)
            ]
            for cp in cps:
                cp.wait()
            pltpu.sync_copy(buf.at[:, pl.ds(0, rows)],
                            g_hbm.at[:, pl.ds(wbase + row0, rows), :])

        def body(ci, carry):
            chunk(ci * _CH, _CH)
            return carry

        lax.fori_loop(0, _NCH, body, 0)
        chunk(_NCH * _CH, _PW - _NCH * _CH)

    return k(all_idx, emb)


def _sc_gather_tail(all_idx, v2):
    info = plsc.get_sparse_core_info()
    nc, ns = info.num_cores, info.num_subcores
    mesh = plsc.VectorSubcoreMesh(core_axis_name="c", subcore_axis_name="s")
    chv = 2 * _CH

    @functools.partial(
        pl.kernel,
        mesh=mesh,
        compiler_params=pltpu.CompilerParams(use_tc_tiling_on_sc=True),
        out_type=jax.ShapeDtypeStruct((_RTOT, 128), jnp.float32),
        scratch_types=[
            pltpu.VMEM((_PW,), jnp.int32),
            pltpu.VMEM((chv, 128), jnp.float32),
            pltpu.SemaphoreType.DMA,
        ],
    )
    def k(aidx_hbm, v_hbm, gt_hbm, idx_v, buf, sem):
        wid = lax.axis_index("s") * nc + lax.axis_index("c")
        wbase = wid * _PW
        pltpu.sync_copy(aidx_hbm.at[pl.ds(wbase, _PW)], idx_v)

        def chunk(row0, rows):
            pltpu.async_copy(
                v_hbm.at[idx_v.at[pl.ds(row0, rows)]],
                buf.at[pl.ds(0, rows)], sem).wait()
            pltpu.sync_copy(buf.at[pl.ds(0, rows)],
                            gt_hbm.at[pl.ds(wbase + row0, rows)])

        def body(ci, carry):
            chunk(ci * chv, chv)
            return carry

        lax.fori_loop(0, _PW // chv, body, 0)
        chunk(_PW - _PW % chv, _PW % chv)

    return k(all_idx, v2)


def _consume_body(g, gt, w1tp, b1, w2t, b2, g1t, a1, packed_ref, mal_ref):
    mm = lambda x, y: jnp.dot(x, y, preferred_element_type=jnp.float32)
    acc = gt[...][:, :H]
    for t in range(_NT):
        acc = acc + mm(g[t], w1tp[t])
    h = jnp.maximum(acc + b1[...], 0.0)
    y = mm(h, w2t[...]) + b2[...]
    z = mm(y, g1t[...])
    z = jnp.where(z >= 0, z, a1[0, 0] * z)

    z3 = z[:8 * S].reshape(8, S, H)
    zroot = z3[:, 0, :]
    pooled = (jnp.sum(z3, axis=1) - zroot) * (1.0 / (S - 1))
    packed_ref[...] = jnp.concatenate([pooled, zroot], axis=1)
    ym = y[8 * S:_PW]
    mal_ref[...] = jnp.concatenate([ym, ym], axis=1)


def _consume(g, gt, w1tp, b1, w2t, b2, g1t, a1):
    full = lambda i: (0, 0)
    out_sds = jax.ShapeDtypeStruct((B, 2 * H), jnp.float32)
    return pl.pallas_call(
        _consume_body,
        grid=(32,),
        in_specs=[
            pl.BlockSpec((_NT, _PW, 128), lambda i: (0, i, 0)),
            pl.BlockSpec((_PW, 128), lambda i: (i, 0)),
            pl.BlockSpec((_NT, 128, H), lambda i: (0, 0, 0)),
            pl.BlockSpec((1, H), full),
            pl.BlockSpec((H, H), full),
            pl.BlockSpec((1, H), full),
            pl.BlockSpec((H, H), full),
            pl.BlockSpec((1, 1), full),
        ],
        out_specs=[
            pl.BlockSpec((8, 2 * H), lambda i: (i, 0)),
            pl.BlockSpec((8, 2 * H), lambda i: (i, 0)),
        ],
        out_shape=[out_sds, out_sds],
    )(g, gt, w1tp, b1, w2t, b2, g1t, a1)


def _score_body(packed, mal, g1t, a1, g2t, a2, vn, vn1, sn, sn1,
                nwt, nb, n1wt, n1b, w1, c1, w2, c2, w3, c3,
                ps_ref, nps_ref, rs_ref, nrs_ref, ms_ref, pool_ref):
    mm = lambda x, y: jnp.dot(x, y, preferred_element_type=jnp.float32)
    pz = packed[...]
    p = pz[:, :H]
    zroot = pz[:, H:]
    m = mal[...][:, H:]

    root = mm(zroot, g2t[...])
    root = jnp.where(root >= 0, root, a2[0, 0] * root)

    vn1h = mm(vn1[...], g1t[...])
    vn1h = jnp.where(vn1h >= 0, vn1h, a1[0, 0] * vn1h)
    u1 = mm(vn1h, w1[...])
    vnh = mm(vn[...], g2t[...])
    vnh = jnp.where(vnh >= 0, vnh, a2[0, 0] * vnh)
    u2 = mm(vnh, w2[...])
    u3 = mm(vnh, w3[...])
    noise = mm(sn1[...], nwt[...]) + nb[...]
    rnoise = mm(sn[...], n1wt[...]) + n1b[...]

    ps = jnp.sum(p * u1, axis=1, keepdims=True) + c1[0, 0]
    ps_ref[...] = ps
    nps_ref[...] = ps + jnp.sum(noise * u1)
    rs = jnp.sum(root * u2, axis=1, keepdims=True) + c2[0, 0]
    rs_ref[...] = rs
    nrs_ref[...] = rs + jnp.sum(rnoise * u2)
    ms_ref[...] = jnp.sum((root + m) * 0.5 * u3, axis=1, keepdims=True) + c3[0, 0]
    pool_ref[...] = p


def _scores(packed, mal, g1t, a1, g2t, a2, vn, vn1, sn, sn1,
            nwt, nb, n1wt, n1b, w1, c1, w2, c2, w3, c3):
    s1 = jax.ShapeDtypeStruct((B, 1), jnp.float32)
    s64 = jax.ShapeDtypeStruct((B, H), jnp.float32)
    return pl.pallas_call(
        _score_body,
        out_shape=[s1, s1, s1, s1, s1, s64],
    )(packed, mal, g1t, a1, g2t, a2, vn, vn1, sn, sn1,
      nwt, nb, n1wt, n1b, w1, c1, w2, c2, w3, c3)


def kernel(subgraph_nodes, edge_index, malicious_nodes, embeddings, fe_W1, fe_b1, fe_W2, fe_b2, g1_W, a1, g2_W, a2, virtual_node, virtual_node1, single_noise, single_noise1, noise_W, noise_b, noise1_W, noise1_b, bil1_W, bil1_b, bil2_W, bil2_b, bil3_W, bil3_b):
    nodes = subgraph_nodes.astype(jnp.int32)
    mal_idx = malicious_nodes.astype(jnp.int32)
    all_idx = jnp.concatenate(
        [nodes.reshape(32, 8 * S), mal_idx.reshape(32, 8)], axis=1).reshape(-1)

    a1r = a1.reshape(1, 1)
    a2r = a2.reshape(1, 1)
    wt = fe_W1.T
    w1tp = wt[:128 * _NT].reshape(_NT, 128, H)
    wtail = jnp.pad(wt[128 * _NT:], ((0, 128 - _TAIL), (0, 0)))

    g = _sc_gather_slabs(all_idx, embeddings)
    v2 = _tail_prepass(embeddings, wtail)
    gt = _sc_gather_tail(all_idx, v2)
    packed, mal = _consume(g, gt, w1tp, fe_b1.reshape(1, H), fe_W2.T,
                           fe_b2.reshape(1, H), g1_W.T, a1r)

    ps, nps, rs, nrs, ms, pooled = _scores(
        packed, mal, g1_W.T, a1r, g2_W.T, a2r,
        virtual_node, virtual_node1, single_noise, single_noise1,
        noise_W.T, noise_b.reshape(1, H), noise1_W.T, noise1_b.reshape(1, H),
        bil1_W[0], bil1_b.reshape(1, 1), bil2_W[0], bil2_b.reshape(1, 1),
        bil3_W[0], bil3_b.reshape(1, 1))
    return (ps, nps, rs, nrs, ms, pooled)


# speedup vs baseline: 3.3810x; 3.3781x over previous
import functools

import jax
import jax.numpy as jnp
from jax import lax
from jax.experimental import pallas as pl
from jax.experimental.pallas import tpu as pltpu
from jax.experimental.pallas import tpu_sc as plsc

N_NODES = 100000
D_FEAT = 1433
B = 256
S = 128
H = 64

_CB = 1024


def _dense_body(embt, w1, b1c, w2, b2c, g1, a1, out_ref):
    x = embt[...]
    h = jnp.maximum(
        jnp.dot(w1[...], x, preferred_element_type=jnp.float32) + b1c[...], 0.0)
    y = jnp.dot(w2[...], h, preferred_element_type=jnp.float32) + b2c[...]
    z = jnp.dot(g1[...], y, preferred_element_type=jnp.float32)
    z = jnp.where(z >= 0, z, a1[0, 0] * z)
    out_ref[...] = jnp.transpose(jnp.concatenate([z, y], axis=0))


def _dense_pass(embt, w1, b1c, w2, b2c, g1, a1):
    n_steps = (N_NODES + _CB - 1) // _CB
    full = lambda i: (0, 0)
    return pl.pallas_call(
        _dense_body,
        grid=(n_steps,),
        in_specs=[
            pl.BlockSpec((D_FEAT, _CB), lambda i: (0, i)),
            pl.BlockSpec((H, D_FEAT), full),
            pl.BlockSpec((H, 1), full),
            pl.BlockSpec((H, H), full),
            pl.BlockSpec((H, 1), full),
            pl.BlockSpec((H, H), full),
            pl.BlockSpec((1, 1), full),
        ],
        out_specs=pl.BlockSpec((_CB, 2 * H), lambda i: (i, 0)),
        out_shape=jax.ShapeDtypeStruct((N_NODES, 2 * H), jnp.float32),
    )(embt, w1, b1c, w2, b2c, g1, a1)


def _sc_gather(nodes, mal_idx, table):
    info = plsc.get_sparse_core_info()
    nc, ns = info.num_cores, info.num_subcores
    nw = nc * ns
    per_w = B // nw
    mesh = plsc.VectorSubcoreMesh(core_axis_name="c", subcore_axis_name="s")
    out_sds = jax.ShapeDtypeStruct((B, 2 * H), jnp.float32)

    @functools.partial(
        pl.kernel,
        mesh=mesh,
        out_type=[out_sds, out_sds],
        scratch_types=[
            pltpu.VMEM((S,), jnp.int32),
            pltpu.VMEM((S, 2 * H), jnp.float32),
            pltpu.VMEM((per_w, 2 * H), jnp.float32),
            pltpu.VMEM((per_w,), jnp.int32),
            pltpu.VMEM((per_w, 2 * H), jnp.float32),
            pltpu.SemaphoreType.DMA,
        ],
    )
    def k(nodes_hbm, midx_hbm, tab_hbm, pooled_hbm, mal_hbm,
          idx_v, rows_v, pool_v, midx_v, mrows_v, sem):
        wid = lax.axis_index("s") * nc + lax.axis_index("c")
        base = wid * per_w

        pltpu.sync_copy(midx_hbm.at[pl.ds(base, per_w)], midx_v)
        pltpu.async_copy(tab_hbm.at[midx_v], mrows_v, sem).wait()
        pltpu.sync_copy(mrows_v, mal_hbm.at[pl.ds(base, per_w)])

        for kk in range(per_w):
            b = base + kk
            pltpu.sync_copy(nodes_hbm.at[b], idx_v)
            pltpu.async_copy(tab_hbm.at[idx_v], rows_v, sem).wait()

            def body(j, acc):
                return tuple(acc[c] + rows_v[j, pl.ds(c * 16, 16)] for c in range(4))

            zero = jnp.zeros((16,), jnp.float32)
            acc = lax.fori_loop(1, S, body, (zero, zero, zero, zero))
            for c in range(4):
                pool_v[kk, pl.ds(c * 16, 16)] = acc[c] * (1.0 / (S - 1))
                pool_v[kk, pl.ds(H + c * 16, 16)] = rows_v[0, pl.ds(c * 16, 16)]
        pltpu.sync_copy(pool_v, pooled_hbm.at[pl.ds(base, per_w)])

    return k(nodes, mal_idx, table)


def _score_body(packed, mal, g1t, a1, g2t, a2, vn, vn1, sn, sn1,
                nwt, nb, n1wt, n1b, w1, c1, w2, c2, w3, c3,
                ps_ref, nps_ref, rs_ref, nrs_ref, ms_ref, pool_ref):
    mm = lambda x, y: jnp.dot(x, y, preferred_element_type=jnp.float32)
    pz = packed[...]
    p = pz[:, :H]
    zroot = pz[:, H:]
    m = mal[...][:, H:]

    root = mm(zroot, g2t[...])
    root = jnp.where(root >= 0, root, a2[0, 0] * root)

    vn1h = mm(vn1[...], g1t[...])
    vn1h = jnp.where(vn1h >= 0, vn1h, a1[0, 0] * vn1h)
    u1 = mm(vn1h, w1[...])
    vnh = mm(vn[...], g2t[...])
    vnh = jnp.where(vnh >= 0, vnh, a2[0, 0] * vnh)
    u2 = mm(vnh, w2[...])
    u3 = mm(vnh, w3[...])
    noise = mm(sn1[...], nwt[...]) + nb[...]
    rnoise = mm(sn[...], n1wt[...]) + n1b[...]

    ps = jnp.sum(p * u1, axis=1, keepdims=True) + c1[0, 0]
    ps_ref[...] = ps
    nps_ref[...] = ps + jnp.sum(noise * u1)
    rs = jnp.sum(root * u2, axis=1, keepdims=True) + c2[0, 0]
    rs_ref[...] = rs
    nrs_ref[...] = rs + jnp.sum(rnoise * u2)
    ms_ref[...] = jnp.sum((root + m) * 0.5 * u3, axis=1, keepdims=True) + c3[0, 0]
    pool_ref[...] = p


def _scores(packed, mal, g1t, a1, g2t, a2, vn, vn1, sn, sn1,
            nwt, nb, n1wt, n1b, w1, c1, w2, c2, w3, c3):
    s1 = jax.ShapeDtypeStruct((B, 1), jnp.float32)
    s64 = jax.ShapeDtypeStruct((B, H), jnp.float32)
    return pl.pallas_call(
        _score_body,
        out_shape=[s1, s1, s1, s1, s1, s64],
    )(packed, mal, g1t, a1, g2t, a2, vn, vn1, sn, sn1,
      nwt, nb, n1wt, n1b, w1, c1, w2, c2, w3, c3)


def kernel(subgraph_nodes, edge_index, malicious_nodes, embeddings, fe_W1, fe_b1, fe_W2, fe_b2, g1_W, a1, g2_W, a2, virtual_node, virtual_node1, single_noise, single_noise1, noise_W, noise_b, noise1_W, noise1_b, bil1_W, bil1_b, bil2_W, bil2_b, bil3_W, bil3_b):
    nodes = subgraph_nodes.astype(jnp.int32)
    mal_idx = malicious_nodes.astype(jnp.int32)

    a1r = a1.reshape(1, 1)
    a2r = a2.reshape(1, 1)

    table = _dense_pass(embeddings.T, fe_W1, fe_b1.reshape(H, 1), fe_W2,
                        fe_b2.reshape(H, 1), g1_W, a1r)
    packed, mal = _sc_gather(nodes, mal_idx, table)

    ps, nps, rs, nrs, ms, pooled = _scores(
        packed, mal, g1_W.T, a1r, g2_W.T, a2r,
        virtual_node, virtual_node1, single_noise, single_noise1,
        noise_W.T, noise_b.reshape(1, H), noise1_W.T, noise1_b.reshape(1, H),
        bil1_W[0], bil1_b.reshape(1, 1), bil2_W[0], bil2_b.reshape(1, 1),
        bil3_W[0], bil3_b.reshape(1, 1))
    return (ps, nps, rs, nrs, ms, pooled)


# speedup vs baseline: 3.6987x; 1.0940x over previous
import functools

import jax
import jax.numpy as jnp
from jax import lax
from jax.experimental import pallas as pl
from jax.experimental.pallas import tpu as pltpu
from jax.experimental.pallas import tpu_sc as plsc

N_NODES = 100000
D_FEAT = 1433
B = 256
S = 128
H = 64

_CB = 2048


def _dense_body(embt, w1, b1c, w2, b2c, g1, a1, out_ref):
    x = embt[...]
    h = jnp.maximum(
        jnp.dot(w1[...], x, preferred_element_type=jnp.float32) + b1c[...], 0.0)
    y = jnp.dot(w2[...], h, preferred_element_type=jnp.float32) + b2c[...]
    z = jnp.dot(g1[...], y, preferred_element_type=jnp.float32)
    z = jnp.where(z >= 0, z, a1[0, 0] * z)
    out_ref[...] = jnp.transpose(jnp.concatenate([z, y], axis=0))


def _dense_pass(embt, w1, b1c, w2, b2c, g1, a1):
    n_steps = (N_NODES + _CB - 1) // _CB
    full = lambda i: (0, 0)
    return pl.pallas_call(
        _dense_body,
        grid=(n_steps,),
        in_specs=[
            pl.BlockSpec((D_FEAT, _CB), lambda i: (0, i)),
            pl.BlockSpec((H, D_FEAT), full),
            pl.BlockSpec((H, 1), full),
            pl.BlockSpec((H, H), full),
            pl.BlockSpec((H, 1), full),
            pl.BlockSpec((H, H), full),
            pl.BlockSpec((1, 1), full),
        ],
        out_specs=pl.BlockSpec((_CB, 2 * H), lambda i: (i, 0)),
        out_shape=jax.ShapeDtypeStruct((N_NODES, 2 * H), jnp.float32),
    )(embt, w1, b1c, w2, b2c, g1, a1)


def _sc_gather(nodes, mal_idx, table):
    info = plsc.get_sparse_core_info()
    nc, ns = info.num_cores, info.num_subcores
    nw = nc * ns
    per_w = B // nw
    mesh = plsc.VectorSubcoreMesh(core_axis_name="c", subcore_axis_name="s")
    out_sds = jax.ShapeDtypeStruct((B, 2 * H), jnp.float32)

    @functools.partial(
        pl.kernel,
        mesh=mesh,
        out_type=[out_sds, out_sds],
        scratch_types=[
            pltpu.VMEM((S,), jnp.int32),
            pltpu.VMEM((S, 2 * H), jnp.float32),
            pltpu.VMEM((per_w, 2 * H), jnp.float32),
            pltpu.VMEM((per_w,), jnp.int32),
            pltpu.VMEM((per_w, 2 * H), jnp.float32),
            pltpu.SemaphoreType.DMA,
        ],
    )
    def k(nodes_hbm, midx_hbm, tab_hbm, pooled_hbm, mal_hbm,
          idx_v, rows_v, pool_v, midx_v, mrows_v, sem):
        wid = lax.axis_index("s") * nc + lax.axis_index("c")
        base = wid * per_w

        pltpu.sync_copy(midx_hbm.at[pl.ds(base, per_w)], midx_v)
        pltpu.async_copy(tab_hbm.at[midx_v], mrows_v, sem).wait()
        pltpu.sync_copy(mrows_v, mal_hbm.at[pl.ds(base, per_w)])

        for kk in range(per_w):
            b = base + kk
            pltpu.sync_copy(nodes_hbm.at[b], idx_v)
            pltpu.async_copy(tab_hbm.at[idx_v], rows_v, sem).wait()

            def body(j, acc):
                return tuple(acc[c] + rows_v[j, pl.ds(c * 16, 16)] for c in range(4))

            zero = jnp.zeros((16,), jnp.float32)
            acc = lax.fori_loop(1, S, body, (zero, zero, zero, zero))
            for c in range(4):
                pool_v[kk, pl.ds(c * 16, 16)] = acc[c] * (1.0 / (S - 1))
                pool_v[kk, pl.ds(H + c * 16, 16)] = rows_v[0, pl.ds(c * 16, 16)]
        pltpu.sync_copy(pool_v, pooled_hbm.at[pl.ds(base, per_w)])

    return k(nodes, mal_idx, table)


def _score_body(packed, mal, g1t, a1, g2t, a2, vn, vn1, sn, sn1,
                nwt, nb, n1wt, n1b, w1, c1, w2, c2, w3, c3,
                ps_ref, nps_ref, rs_ref, nrs_ref, ms_ref, pool_ref):
    mm = lambda x, y: jnp.dot(x, y, preferred_element_type=jnp.float32)
    pz = packed[...]
    p = pz[:, :H]
    zroot = pz[:, H:]
    m = mal[...][:, H:]

    root = mm(zroot, g2t[...])
    root = jnp.where(root >= 0, root, a2[0, 0] * root)

    vn1h = mm(vn1[...], g1t[...])
    vn1h = jnp.where(vn1h >= 0, vn1h, a1[0, 0] * vn1h)
    u1 = mm(vn1h, w1[...])
    vnh = mm(vn[...], g2t[...])
    vnh = jnp.where(vnh >= 0, vnh, a2[0, 0] * vnh)
    u2 = mm(vnh, w2[...])
    u3 = mm(vnh, w3[...])
    noise = mm(sn1[...], nwt[...]) + nb[...]
    rnoise = mm(sn[...], n1wt[...]) + n1b[...]

    ps = jnp.sum(p * u1, axis=1, keepdims=True) + c1[0, 0]
    ps_ref[...] = ps
    nps_ref[...] = ps + jnp.sum(noise * u1)
    rs = jnp.sum(root * u2, axis=1, keepdims=True) + c2[0, 0]
    rs_ref[...] = rs
    nrs_ref[...] = rs + jnp.sum(rnoise * u2)
    ms_ref[...] = jnp.sum((root + m) * 0.5 * u3, axis=1, keepdims=True) + c3[0, 0]
    pool_ref[...] = p


def _scores(packed, mal, g1t, a1, g2t, a2, vn, vn1, sn, sn1,
            nwt, nb, n1wt, n1b, w1, c1, w2, c2, w3, c3):
    s1 = jax.ShapeDtypeStruct((B, 1), jnp.float32)
    s64 = jax.ShapeDtypeStruct((B, H), jnp.float32)
    return pl.pallas_call(
        _score_body,
        out_shape=[s1, s1, s1, s1, s1, s64],
    )(packed, mal, g1t, a1, g2t, a2, vn, vn1, sn, sn1,
      nwt, nb, n1wt, n1b, w1, c1, w2, c2, w3, c3)


def kernel(subgraph_nodes, edge_index, malicious_nodes, embeddings, fe_W1, fe_b1, fe_W2, fe_b2, g1_W, a1, g2_W, a2, virtual_node, virtual_node1, single_noise, single_noise1, noise_W, noise_b, noise1_W, noise1_b, bil1_W, bil1_b, bil2_W, bil2_b, bil3_W, bil3_b):
    nodes = subgraph_nodes.astype(jnp.int32)
    mal_idx = malicious_nodes.astype(jnp.int32)

    a1r = a1.reshape(1, 1)
    a2r = a2.reshape(1, 1)

    table = _dense_pass(embeddings.T, fe_W1, fe_b1.reshape(H, 1), fe_W2,
                        fe_b2.reshape(H, 1), g1_W, a1r)
    packed, mal = _sc_gather(nodes, mal_idx, table)

    ps, nps, rs, nrs, ms, pooled = _scores(
        packed, mal, g1_W.T, a1r, g2_W.T, a2r,
        virtual_node, virtual_node1, single_noise, single_noise1,
        noise_W.T, noise_b.reshape(1, H), noise1_W.T, noise1_b.reshape(1, H),
        bil1_W[0], bil1_b.reshape(1, 1), bil2_W[0], bil2_b.reshape(1, 1),
        bil3_W[0], bil3_b.reshape(1, 1))
    return (ps, nps, rs, nrs, ms, pooled)


# speedup vs baseline: 3.7038x; 1.0014x over previous
import functools

import jax
import jax.numpy as jnp
from jax import lax
from jax.experimental import pallas as pl
from jax.experimental.pallas import tpu as pltpu
from jax.experimental.pallas import tpu_sc as plsc

N_NODES = 100000
D_FEAT = 1433
B = 256
S = 128
H = 64

_CB = 4096


def _dense_body(embt, w1, b1c, w2, b2c, g1, a1, out_ref):
    x = embt[...]
    h = jnp.maximum(
        jnp.dot(w1[...], x, preferred_element_type=jnp.float32) + b1c[...], 0.0)
    y = jnp.dot(w2[...], h, preferred_element_type=jnp.float32) + b2c[...]
    z = jnp.dot(g1[...], y, preferred_element_type=jnp.float32)
    z = jnp.where(z >= 0, z, a1[0, 0] * z)
    out_ref[...] = jnp.transpose(jnp.concatenate([z, y], axis=0))


def _dense_pass(embt, w1, b1c, w2, b2c, g1, a1):
    n_steps = (N_NODES + _CB - 1) // _CB
    full = lambda i: (0, 0)
    return pl.pallas_call(
        _dense_body,
        grid=(n_steps,),
        in_specs=[
            pl.BlockSpec((D_FEAT, _CB), lambda i: (0, i)),
            pl.BlockSpec((H, D_FEAT), full),
            pl.BlockSpec((H, 1), full),
            pl.BlockSpec((H, H), full),
            pl.BlockSpec((H, 1), full),
            pl.BlockSpec((H, H), full),
            pl.BlockSpec((1, 1), full),
        ],
        out_specs=pl.BlockSpec((_CB, 2 * H), lambda i: (i, 0)),
        out_shape=jax.ShapeDtypeStruct((N_NODES, 2 * H), jnp.float32),
    )(embt, w1, b1c, w2, b2c, g1, a1)


def _sc_gather(nodes, mal_idx, table):
    info = plsc.get_sparse_core_info()
    nc, ns = info.num_cores, info.num_subcores
    nw = nc * ns
    per_w = B // nw
    mesh = plsc.VectorSubcoreMesh(core_axis_name="c", subcore_axis_name="s")
    out_sds = jax.ShapeDtypeStruct((B, 2 * H), jnp.float32)

    @functools.partial(
        pl.kernel,
        mesh=mesh,
        out_type=[out_sds, out_sds],
        scratch_types=[
            pltpu.VMEM((S,), jnp.int32),
            pltpu.VMEM((S, 2 * H), jnp.float32),
            pltpu.VMEM((per_w, 2 * H), jnp.float32),
            pltpu.VMEM((per_w,), jnp.int32),
            pltpu.VMEM((per_w, 2 * H), jnp.float32),
            pltpu.SemaphoreType.DMA,
        ],
    )
    def k(nodes_hbm, midx_hbm, tab_hbm, pooled_hbm, mal_hbm,
          idx_v, rows_v, pool_v, midx_v, mrows_v, sem):
        wid = lax.axis_index("s") * nc + lax.axis_index("c")
        base = wid * per_w

        pltpu.sync_copy(midx_hbm.at[pl.ds(base, per_w)], midx_v)
        pltpu.async_copy(tab_hbm.at[midx_v], mrows_v, sem).wait()
        pltpu.sync_copy(mrows_v, mal_hbm.at[pl.ds(base, per_w)])

        for kk in range(per_w):
            b = base + kk
            pltpu.sync_copy(nodes_hbm.at[b], idx_v)
            pltpu.async_copy(tab_hbm.at[idx_v], rows_v, sem).wait()

            def body(j, acc):
                return tuple(acc[c] + rows_v[j, pl.ds(c * 16, 16)] for c in range(4))

            zero = jnp.zeros((16,), jnp.float32)
            acc = lax.fori_loop(1, S, body, (zero, zero, zero, zero))
            for c in range(4):
                pool_v[kk, pl.ds(c * 16, 16)] = acc[c] * (1.0 / (S - 1))
                pool_v[kk, pl.ds(H + c * 16, 16)] = rows_v[0, pl.ds(c * 16, 16)]
        pltpu.sync_copy(pool_v, pooled_hbm.at[pl.ds(base, per_w)])

    return k(nodes, mal_idx, table)


def _score_body(packed, mal, g1t, a1, g2t, a2, vn, vn1, sn, sn1,
                nwt, nb, n1wt, n1b, w1, c1, w2, c2, w3, c3,
                ps_ref, nps_ref, rs_ref, nrs_ref, ms_ref, pool_ref):
    mm = lambda x, y: jnp.dot(x, y, preferred_element_type=jnp.float32)
    pz = packed[...]
    p = pz[:, :H]
    zroot = pz[:, H:]
    m = mal[...][:, H:]

    root = mm(zroot, g2t[...])
    root = jnp.where(root >= 0, root, a2[0, 0] * root)

    vn1h = mm(vn1[...], g1t[...])
    vn1h = jnp.where(vn1h >= 0, vn1h, a1[0, 0] * vn1h)
    u1 = mm(vn1h, w1[...])
    vnh = mm(vn[...], g2t[...])
    vnh = jnp.where(vnh >= 0, vnh, a2[0, 0] * vnh)
    u2 = mm(vnh, w2[...])
    u3 = mm(vnh, w3[...])
    noise = mm(sn1[...], nwt[...]) + nb[...]
    rnoise = mm(sn[...], n1wt[...]) + n1b[...]

    ps = jnp.sum(p * u1, axis=1, keepdims=True) + c1[0, 0]
    ps_ref[...] = ps
    nps_ref[...] = ps + jnp.sum(noise * u1)
    rs = jnp.sum(root * u2, axis=1, keepdims=True) + c2[0, 0]
    rs_ref[...] = rs
    nrs_ref[...] = rs + jnp.sum(rnoise * u2)
    ms_ref[...] = jnp.sum((root + m) * 0.5 * u3, axis=1, keepdims=True) + c3[0, 0]
    pool_ref[...] = p


def _scores(packed, mal, g1t, a1, g2t, a2, vn, vn1, sn, sn1,
            nwt, nb, n1wt, n1b, w1, c1, w2, c2, w3, c3):
    s1 = jax.ShapeDtypeStruct((B, 1), jnp.float32)
    s64 = jax.ShapeDtypeStruct((B, H), jnp.float32)
    return pl.pallas_call(
        _score_body,
        out_shape=[s1, s1, s1, s1, s1, s64],
    )(packed, mal, g1t, a1, g2t, a2, vn, vn1, sn, sn1,
      nwt, nb, n1wt, n1b, w1, c1, w2, c2, w3, c3)


def kernel(subgraph_nodes, edge_index, malicious_nodes, embeddings, fe_W1, fe_b1, fe_W2, fe_b2, g1_W, a1, g2_W, a2, virtual_node, virtual_node1, single_noise, single_noise1, noise_W, noise_b, noise1_W, noise1_b, bil1_W, bil1_b, bil2_W, bil2_b, bil3_W, bil3_b):
    nodes = subgraph_nodes.astype(jnp.int32)
    mal_idx = malicious_nodes.astype(jnp.int32)

    a1r = a1.reshape(1, 1)
    a2r = a2.reshape(1, 1)

    table = _dense_pass(embeddings.T, fe_W1, fe_b1.reshape(H, 1), fe_W2,
                        fe_b2.reshape(H, 1), g1_W, a1r)
    packed, mal = _sc_gather(nodes, mal_idx, table)

    ps, nps, rs, nrs, ms, pooled = _scores(
        packed, mal, g1_W.T, a1r, g2_W.T, a2r,
        virtual_node, virtual_node1, single_noise, single_noise1,
        noise_W.T, noise_b.reshape(1, H), noise1_W.T, noise1_b.reshape(1, H),
        bil1_W[0], bil1_b.reshape(1, 1), bil2_W[0], bil2_b.reshape(1, 1),
        bil3_W[0], bil3_b.reshape(1, 1))
    return (ps, nps, rs, nrs, ms, pooled)
